# baseline (speedup 1.0000x reference)
import functools

import jax
import jax.numpy as jnp
from jax import lax
from jax.experimental import pallas as pl
from jax.experimental.pallas import tpu as pltpu
from jax.experimental.pallas import tpu_sc as plsc

N_NODES = 10000
N_EDGES = 320000
D = 128
NUM_REL = 237

_INFO = plsc.get_sparse_core_info()
NC = _INFO.num_cores
NS = _INFO.num_subcores
NW = NC * NS
LANES = 16

E_PER_W = N_EDGES // NW
CHUNK = 80
N_ITER = E_PER_W // CHUNK
N_PAIR = (N_ITER - 1) // 2
GROUPS = CHUNK // LANES


def _make_kernel():
    mesh = plsc.VectorSubcoreMesh(core_axis_name="c", subcore_axis_name="s")

    @functools.partial(
        pl.kernel,
        out_type=jax.ShapeDtypeStruct((N_EDGES,), jnp.float32),
        mesh=mesh,
        compiler_params=pltpu.CompilerParams(needs_layout_passes=False),
        scratch_types=[
            pltpu.VMEM((E_PER_W,), jnp.int32),
            pltpu.VMEM((E_PER_W,), jnp.int32),
            pltpu.VMEM((E_PER_W,), jnp.int32),
            pltpu.VMEM((NUM_REL, D), jnp.float32),
            pltpu.VMEM((CHUNK, D), jnp.float32),
            pltpu.VMEM((CHUNK, D), jnp.float32),
            pltpu.VMEM((CHUNK, D), jnp.float32),
            pltpu.VMEM((CHUNK, D), jnp.float32),
            pltpu.VMEM((E_PER_W,), jnp.float32),
            pltpu.SemaphoreType.DMA,
            pltpu.SemaphoreType.DMA,
        ],
    )
    def k(x_hbm, h_hbm, t_hbm, et_hbm, rel_hbm, out_hbm,
          hidx, tidx, etidx, reltab, bh0, bh1, bt0, bt1, outbuf,
          sem0, sem1):
        wid = lax.axis_index("s") * NC + lax.axis_index("c")
        wbase = wid * E_PER_W

        iota = lax.iota(jnp.int32, 16)
        bufs = ((bh0, bt0, sem0), (bh1, bt1, sem1))

        pltpu.sync_copy(h_hbm.at[pl.ds(wbase, E_PER_W)], hidx)
        pltpu.sync_copy(t_hbm.at[pl.ds(wbase, E_PER_W)], tidx)
        pltpu.sync_copy(et_hbm.at[pl.ds(wbase, E_PER_W)], etidx)
        pltpu.sync_copy(rel_hbm, reltab)

        def fire(i, slot):
            bh, bt, sem = bufs[slot]
            sl = pl.ds(i * CHUNK, CHUNK)
            pltpu.async_copy(x_hbm.at[hidx.at[sl]], bh, sem)
            pltpu.async_copy(x_hbm.at[tidx.at[sl]], bt, sem)

        def drain(slot):
            bh, bt, sem = bufs[slot]
            for b in (bh, bt):
                pltpu.make_async_copy(
                    x_hbm.at[hidx.at[pl.ds(0, CHUNK)]], b, sem
                ).wait()

        def compute(i, slot):
            bh, bt, _ = bufs[slot]

            def grp(g, _):
                off = i * CHUNK + g * LANES
                rows = iota + g * LANES
                etv = etidx[pl.ds(off, LANES)]
                acc = jnp.zeros((LANES,), jnp.float32)
                cols = jnp.zeros((LANES,), jnp.int32)
                one = jnp.ones((LANES,), jnp.int32)
                for c in range(D):
                    v = (plsc.load_gather(bh, [rows, cols])
                         + plsc.load_gather(reltab, [etv, cols])
                         - plsc.load_gather(bt, [rows, cols]))
                    acc = acc + jnp.abs(v)
                    if c + 1 < D:
                        cols = cols + one
                outbuf[pl.ds(off, LANES)] = acc
                return ()

            lax.fori_loop(0, GROUPS, grp, (), unroll=False)

        fire(0, 0)

        def pair(p, _):
            i0 = 2 * p
            drain(0)
            fire(i0 + 1, 1)
            compute(i0, 0)
            drain(1)
            fire(i0 + 2, 0)
            compute(i0 + 1, 1)
            return ()

        lax.fori_loop(0, N_PAIR, pair, (), unroll=False)

        drain(0)
        compute(N_ITER - 1, 0)

        pltpu.sync_copy(outbuf, out_hbm.at[pl.ds(wbase, E_PER_W)])

    return k


_kernel_call = _make_kernel()


@jax.jit
def kernel(x, edge_index, edge_type, rel_embedding):
    h = edge_index[0]
    t = edge_index[1]
    return _kernel_call(x, h, t, edge_type, rel_embedding)


# speedup vs baseline: 2.9625x; 2.9625x over previous
import functools

import jax
import jax.numpy as jnp
from jax import lax
from jax.experimental import pallas as pl
from jax.experimental.pallas import tpu as pltpu
from jax.experimental.pallas import tpu_sc as plsc

N_NODES = 10000
N_EDGES = 320000
D = 128
NUM_REL = 237

_INFO = plsc.get_sparse_core_info()
NC = _INFO.num_cores
NS = _INFO.num_subcores
NW = NC * NS
LANES = 16
VPR = D // LANES

E_PER_W = N_EDGES // NW
CHUNK = 80
N_ITER = E_PER_W // CHUNK
N_PAIR = (N_ITER - 1) // 2
GROUPS = CHUNK // LANES


def _make_kernel():
    mesh = plsc.VectorSubcoreMesh(core_axis_name="c", subcore_axis_name="s")

    @functools.partial(
        pl.kernel,
        out_type=jax.ShapeDtypeStruct((N_EDGES,), jnp.float32),
        mesh=mesh,
        compiler_params=pltpu.CompilerParams(needs_layout_passes=False),
        scratch_types=[
            pltpu.VMEM((E_PER_W,), jnp.int32),
            pltpu.VMEM((CHUNK,), jnp.int32),
            pltpu.VMEM((CHUNK,), jnp.int32),
            pltpu.VMEM((CHUNK,), jnp.int32),
            pltpu.VMEM((CHUNK,), jnp.int32),
            pltpu.VMEM((NUM_REL, D), jnp.float32),
            pltpu.VMEM((CHUNK, D), jnp.float32),
            pltpu.VMEM((CHUNK, D), jnp.float32),
            pltpu.VMEM((CHUNK, D), jnp.float32),
            pltpu.VMEM((CHUNK, D), jnp.float32),
            pltpu.VMEM((E_PER_W,), jnp.float32),
            pltpu.SemaphoreType.DMA,
            pltpu.SemaphoreType.DMA,
            pltpu.SemaphoreType.DMA,
            pltpu.SemaphoreType.DMA,
        ],
    )
    def k(x_hbm, h_hbm, t_hbm, et_hbm, rel_hbm, out_hbm,
          etidx, hi0, hi1, ti0, ti1, reltab, bh0, bh1, bt0, bt1, outbuf,
          semi0, semi1, semr0, semr1):
        wid = lax.axis_index("s") * NC + lax.axis_index("c")
        wbase = wid * E_PER_W

        lane = lax.iota(jnp.int32, 16)
        ibufs = ((hi0, ti0, semi0), (hi1, ti1, semi1))
        rbufs = ((bh0, bt0, semr0), (bh1, bt1, semr1))

        pltpu.sync_copy(et_hbm.at[pl.ds(wbase, E_PER_W)], etidx)
        pltpu.sync_copy(rel_hbm, reltab)

        def fire_idx(i, slot):
            hi, ti, sem = ibufs[slot]
            sl = pl.ds(wbase + i * CHUNK, CHUNK)
            pltpu.async_copy(h_hbm.at[sl], hi, sem)
            pltpu.async_copy(t_hbm.at[sl], ti, sem)

        def drain_idx(slot):
            hi, ti, sem = ibufs[slot]
            for b in (hi, ti):
                pltpu.make_async_copy(h_hbm.at[pl.ds(0, CHUNK)], b, sem).wait()

        def fire_rows(slot):
            hi, ti, _ = ibufs[slot]
            bh, bt, sem = rbufs[slot]
            pltpu.async_copy(x_hbm.at[hi], bh, sem)
            pltpu.async_copy(x_hbm.at[ti], bt, sem)

        def drain_rows(slot):
            hi, _, _ = ibufs[slot]
            bh, bt, sem = rbufs[slot]
            for b in (bh, bt):
                pltpu.make_async_copy(x_hbm.at[hi], b, sem).wait()

        def compute(i, slot):
            bh, bt, _ = rbufs[slot]
            obase = i * CHUNK
            for g in range(GROUPS):
                etv = etidx[pl.ds(obase + g * LANES, LANES)]
                scores = jnp.zeros((LANES,), jnp.float32)
                for j in range(LANES):
                    e = g * LANES + j
                    rrow = reltab.at[etv[j]]
                    acc = None
                    for kk in range(VPR):
                        sl = pl.ds(kk * LANES, LANES)
                        v = bh[e, sl] + rrow[sl] - bt[e, sl]
                        a = jnp.abs(v)
                        acc = a if acc is None else acc + a
                    tot = jnp.sum(acc)
                    scores = jnp.where(lane == j, tot, scores)
                outbuf[pl.ds(obase + g * LANES, LANES)] = scores

        def compute_dyn(i, slot):
            bh, bt, _ = rbufs[slot]
            obase = i * CHUNK

            def grp(g, _):
                etv = etidx[pl.ds(obase + g * LANES, LANES)]
                scores = jnp.zeros((LANES,), jnp.float32)
                for j in range(LANES):
                    rrow = reltab.at[etv[j]]
                    acc = None
                    for kk in range(VPR):
                        sl = pl.ds(kk * LANES, LANES)
                        v = bh[g * LANES + j, sl] + rrow[sl] - bt[g * LANES + j, sl]
                        a = jnp.abs(v)
                        acc = a if acc is None else acc + a
                    tot = jnp.sum(acc)
                    scores = jnp.where(lane == j, tot, scores)
                outbuf[pl.ds(obase + g * LANES, LANES)] = scores
                return ()

            lax.fori_loop(0, GROUPS, grp, (), unroll=False)

        fire_idx(0, 0)
        fire_idx(1, 1)
        drain_idx(0)
        fire_rows(0)

        def pair(p, _):
            i0 = 2 * p
            drain_rows(0)
            fire_idx(i0 + 2, 0)
            drain_idx(1)
            fire_rows(1)
            compute(i0, 0)
            drain_rows(1)
            @pl.when(i0 + 3 < N_ITER)
            def _():
                fire_idx(i0 + 3, 1)
            drain_idx(0)
            fire_rows(0)
            compute(i0 + 1, 1)
            return ()

        lax.fori_loop(0, N_PAIR, pair, (), unroll=False)

        drain_rows(0)
        compute_dyn(N_ITER - 1, 0)

        pltpu.sync_copy(outbuf, out_hbm.at[pl.ds(wbase, E_PER_W)])

    return k


_kernel_call = _make_kernel()


@jax.jit
def kernel(x, edge_index, edge_type, rel_embedding):
    h = edge_index[0]
    t = edge_index[1]
    return _kernel_call(x, h, t, edge_type, rel_embedding)
